# baseline (speedup 1.0000x reference)
import functools

import jax
import jax.numpy as jnp
from jax import lax
from jax.experimental import pallas as pl
from jax.experimental.pallas import tpu as pltpu
from jax.experimental.pallas import tpu_sc as plsc

_NUM_EMB = 100000
_DIM = 64
_B = 4096
_L = 50
_NW = 32
_SPW = _B // _NW
_PAD = _DIM + 1


def _build():
    mesh = plsc.VectorSubcoreMesh(core_axis_name="c", subcore_axis_name="s")

    @functools.partial(
        pl.kernel,
        mesh=mesh,
        out_type=jax.ShapeDtypeStruct((_L, 8, _NW, 8, 128), jnp.float32),
        scratch_types=[
            pltpu.VMEM((_SPW, _L), jnp.int32),
            pltpu.VMEM((_L, _SPW), jnp.int32),
            pltpu.VMEM((_SPW, _DIM), jnp.float32),
            pltpu.VMEM((_SPW, _DIM), jnp.float32),
            pltpu.VMEM((_SPW, _PAD), jnp.float32),
            pltpu.VMEM((_SPW, _PAD), jnp.float32),
            pltpu.VMEM((8, 8, 128), jnp.float32),
            pltpu.VMEM((8, 8, 128), jnp.float32),
            [pltpu.SemaphoreType.DMA] * 2,
            [pltpu.SemaphoreType.DMA] * 2,
        ],
        compiler_params=pltpu.CompilerParams(
            use_tc_tiling_on_sc=False, needs_layout_passes=False
        ),
    )
    def gather_kernel(idx_hbm, table_hbm, out_hbm,
                      idx_v, idx_t, g0, g1, s0, s1, t0, t1, gsems, ssems):
        wid = lax.axis_index("s") * 2 + lax.axis_index("c")
        base = wid * _SPW
        pltpu.sync_copy(idx_hbm.at[pl.ds(base, _SPW)], idx_v)

        iota = lax.iota(jnp.int32, 16)
        rows = [iota + (16 * j) for j in range(8)]
        gbufs = (g0, g1)
        sbufs = (s0, s1)
        tbufs = (t0, t1)

        @plsc.parallel_loop(0, _L)
        def tr_idx(l):
            col = jnp.zeros((16,), jnp.int32) + l
            for j in range(8):
                v = plsc.load_gather(idx_v, [rows[j], col])
                idx_t[l, pl.ds(16 * j, 16)] = v

        def fire_g(l, b):
            pltpu.async_copy(table_hbm.at[idx_t.at[l]], gbufs[b], gsems[b])

        def drain_g(l, b):
            pltpu.make_async_copy(
                table_hbm.at[idx_t.at[l]], gbufs[b], gsems[b]
            ).wait()

        def fire_s(l, b):
            pltpu.async_copy(tbufs[b], out_hbm.at[l].at[:, wid], ssems[b])

        def drain_s(b):
            pltpu.make_async_copy(
                tbufs[b], out_hbm.at[0].at[:, wid], ssems[b]
            ).wait()

        def transpose(b):
            g = gbufs[b]
            s = sbufs[b]
            t = tbufs[b]

            @plsc.parallel_loop(0, _SPW // 8, unroll=2)
            def cp(i):
                for jj in range(8):
                    r = i * 8 + jj
                    for c in range(4):
                        s[r, pl.ds(16 * c, 16)] = g[r, pl.ds(16 * c, 16)]

            @plsc.parallel_loop(0, 8, unroll=2)
            def tr(dh):
                for dl in range(8):
                    col = jnp.zeros((16,), jnp.int32) + (dh * 8 + dl)
                    for j in range(8):
                        v = plsc.load_gather(s, [rows[j], col])
                        t[dh, dl, pl.ds(16 * j, 16)] = v

        fire_g(0, 0)

        def body(i, carry):
            for d in range(2):
                l = i * 2 + d

                @pl.when(l + 1 < _L)
                def _():
                    fire_g(l + 1, 1 - d)

                drain_g(l, d)

                @pl.when(l >= 2)
                def _():
                    drain_s(d)

                transpose(d)
                fire_s(l, d)
            return carry

        lax.fori_loop(0, _L // 2, body, 0)
        drain_s(0)
        drain_s(1)

    return gather_kernel


_gather_cache = []


def kernel(item_seq_batch, table):
    if not _gather_cache:
        _gather_cache.append(_build())
    idx = item_seq_batch.astype(jnp.int32)
    out5 = _gather_cache[0](idx, table)
    return jnp.transpose(out5, (2, 4, 0, 1, 3)).reshape(_B, _L, _DIM)


# speedup vs baseline: 1.2990x; 1.2990x over previous
import functools

import jax
import jax.numpy as jnp
from jax import lax
from jax.experimental import pallas as pl
from jax.experimental.pallas import tpu as pltpu
from jax.experimental.pallas import tpu_sc as plsc

_NUM_EMB = 100000
_DIM = 64
_B = 4096
_L = 50
_NW = 32
_SPW = _B // _NW
_PAD = _DIM + 1


def _build():
    mesh = plsc.VectorSubcoreMesh(core_axis_name="c", subcore_axis_name="s")

    @functools.partial(
        pl.kernel,
        mesh=mesh,
        out_type=jax.ShapeDtypeStruct((_L, 8, _NW, 8, 128), jnp.float32),
        scratch_types=[
            pltpu.VMEM((_SPW, _L), jnp.int32),
            pltpu.VMEM((_L, _SPW), jnp.int32),
            pltpu.VMEM((_SPW, _DIM), jnp.float32),
            pltpu.VMEM((_SPW, _DIM), jnp.float32),
            pltpu.VMEM((_SPW * _PAD,), jnp.float32),
            pltpu.VMEM((_SPW * _PAD,), jnp.float32),
            pltpu.VMEM((8, 8, 128), jnp.float32),
            pltpu.VMEM((8, 8, 128), jnp.float32),
            [pltpu.SemaphoreType.DMA] * 2,
            [pltpu.SemaphoreType.DMA] * 2,
        ],
        compiler_params=pltpu.CompilerParams(
            use_tc_tiling_on_sc=False, needs_layout_passes=False
        ),
    )
    def gather_kernel(idx_hbm, table_hbm, out_hbm,
                      idx_v, idx_t, g0, g1, s0, s1, t0, t1, gsems, ssems):
        wid = lax.axis_index("s") * 2 + lax.axis_index("c")
        base = wid * _SPW
        pltpu.sync_copy(idx_hbm.at[pl.ds(base, _SPW)], idx_v)

        iota = lax.iota(jnp.int32, 16)
        rows = [iota + (16 * j) for j in range(8)]
        rows65 = [(iota + (16 * j)) * _PAD for j in range(8)]
        gbufs = (g0, g1)
        sbufs = (s0, s1)
        tbufs = (t0, t1)

        @plsc.parallel_loop(0, _L)
        def tr_idx(l):
            col = jnp.zeros((16,), jnp.int32) + l
            for j in range(8):
                v = plsc.load_gather(idx_v, [rows[j], col])
                idx_t[l, pl.ds(16 * j, 16)] = v

        def fire_g(l, b):
            pltpu.async_copy(table_hbm.at[idx_t.at[l]], gbufs[b], gsems[b])

        def drain_g(l, b):
            pltpu.make_async_copy(
                table_hbm.at[idx_t.at[l]], gbufs[b], gsems[b]
            ).wait()

        def fire_s(l, b):
            pltpu.async_copy(tbufs[b], out_hbm.at[l].at[:, wid], ssems[b])

        def drain_s(b):
            pltpu.make_async_copy(
                tbufs[b], out_hbm.at[0].at[:, wid], ssems[b]
            ).wait()

        def transpose(b):
            g = gbufs[b]
            s = sbufs[b]
            t = tbufs[b]

            @plsc.parallel_loop(0, _SPW // 8)
            def cp(i):
                for jj in range(8):
                    r = i * 8 + jj
                    for c in range(4):
                        s[pl.ds(r * _PAD + 16 * c, 16)] = g[r, pl.ds(16 * c, 16)]

            @plsc.parallel_loop(0, 8)
            def tr(dh):
                for dl in range(8):
                    d = dh * 8 + dl
                    for j in range(8):
                        v = plsc.load_gather(s, [rows65[j] + d])
                        t[dh, dl, pl.ds(16 * j, 16)] = v

        fire_g(0, 0)

        def body(i, carry):
            for d in range(2):
                l = i * 2 + d

                @pl.when(l + 1 < _L)
                def _():
                    fire_g(l + 1, 1 - d)

                drain_g(l, d)

                @pl.when(l >= 2)
                def _():
                    drain_s(d)

                transpose(d)
                fire_s(l, d)
            return carry

        lax.fori_loop(0, _L // 2, body, 0)
        drain_s(0)
        drain_s(1)

    return gather_kernel


_gather_cache = []


def kernel(item_seq_batch, table):
    if not _gather_cache:
        _gather_cache.append(_build())
    idx = item_seq_batch.astype(jnp.int32)
    out5 = _gather_cache[0](idx, table)
    return jnp.transpose(out5, (2, 4, 0, 1, 3)).reshape(_B, _L, _DIM)


# speedup vs baseline: 1.3153x; 1.0125x over previous
import functools

import jax
import jax.numpy as jnp
from jax import lax
from jax.experimental import pallas as pl
from jax.experimental.pallas import tpu as pltpu
from jax.experimental.pallas import tpu_sc as plsc

_NUM_EMB = 100000
_DIM = 64
_B = 4096
_L = 50
_NW = 32
_SPW = _B // _NW
_PAD = _DIM + 1


def _build():
    mesh = plsc.VectorSubcoreMesh(core_axis_name="c", subcore_axis_name="s")

    @functools.partial(
        pl.kernel,
        mesh=mesh,
        out_type=jax.ShapeDtypeStruct((_L, 8, _NW, 8, 128), jnp.float32),
        scratch_types=[
            pltpu.VMEM((_L, _SPW), jnp.int32),
            pltpu.VMEM((_SPW, _DIM), jnp.float32),
            pltpu.VMEM((_SPW, _DIM), jnp.float32),
            pltpu.VMEM((_SPW * _PAD,), jnp.float32),
            pltpu.VMEM((_SPW * _PAD,), jnp.float32),
            pltpu.VMEM((8, 8, 128), jnp.float32),
            pltpu.VMEM((8, 8, 128), jnp.float32),
            [pltpu.SemaphoreType.DMA] * 2,
            [pltpu.SemaphoreType.DMA] * 2,
        ],
        compiler_params=pltpu.CompilerParams(
            use_tc_tiling_on_sc=False, needs_layout_passes=False
        ),
    )
    def gather_kernel(idx_hbm, table_hbm, out_hbm,
                      idx_t, g0, g1, s0, s1, t0, t1, gsems, ssems):
        wid = lax.axis_index("s") * 2 + lax.axis_index("c")
        base = wid * _SPW
        pltpu.sync_copy(idx_hbm.at[:, pl.ds(base, _SPW)], idx_t)

        iota = lax.iota(jnp.int32, 16)
        rows65 = [(iota + (16 * j)) * _PAD for j in range(8)]
        gbufs = (g0, g1)
        sbufs = (s0, s1)
        tbufs = (t0, t1)

        def fire_g(l, b):
            pltpu.async_copy(table_hbm.at[idx_t.at[l]], gbufs[b], gsems[b])

        def drain_g(l, b):
            pltpu.make_async_copy(
                table_hbm.at[idx_t.at[l]], gbufs[b], gsems[b]
            ).wait()

        def fire_s(l, b):
            pltpu.async_copy(tbufs[b], out_hbm.at[l].at[:, wid], ssems[b])

        def drain_s(b):
            pltpu.make_async_copy(
                tbufs[b], out_hbm.at[0].at[:, wid], ssems[b]
            ).wait()

        def transpose(b):
            g = gbufs[b]
            s = sbufs[b]
            t = tbufs[b]

            @plsc.parallel_loop(0, _SPW // 8)
            def cp(i):
                for jj in range(8):
                    r = i * 8 + jj
                    for c in range(4):
                        s[pl.ds(r * _PAD + 16 * c, 16)] = g[r, pl.ds(16 * c, 16)]

            @plsc.parallel_loop(0, 8)
            def tr(dh):
                for dl in range(8):
                    d = dh * 8 + dl
                    for j in range(8):
                        v = plsc.load_gather(s, [rows65[j] + d])
                        t[dh, dl, pl.ds(16 * j, 16)] = v

        fire_g(0, 0)

        def body(i, carry):
            for d in range(2):
                l = i * 2 + d

                @pl.when(l + 1 < _L)
                def _():
                    fire_g(l + 1, 1 - d)

                drain_g(l, d)

                @pl.when(l >= 2)
                def _():
                    drain_s(d)

                transpose(d)
                fire_s(l, d)
            return carry

        lax.fori_loop(0, _L // 2, body, 0)
        drain_s(0)
        drain_s(1)

    return gather_kernel


_gather_cache = []


def kernel(item_seq_batch, table):
    if not _gather_cache:
        _gather_cache.append(_build())
    idx = jnp.transpose(item_seq_batch.astype(jnp.int32))
    out5 = _gather_cache[0](idx, table)
    return jnp.transpose(out5, (2, 4, 0, 1, 3)).reshape(_B, _L, _DIM)
